# Initial kernel scaffold; baseline (speedup 1.0000x reference)
import jax
import jax.numpy as jnp
from jax import lax
from jax.experimental import pallas as pl
from jax.experimental.pallas import tpu as pltpu
from jax.experimental.pallas import tpu_sc as plsc

VOCAB = 1_000_000
L = 200
D = 32
B = 4096
N = B * L

NC, NS = 2, 16
NW = NC * NS
PER_W = N // NW

IW = 128
C = 1280
K = C // IW
NCHUNK = PER_W // C


def _emb_body(x_hbm, tok_hbm, pos_hbm, out_hbm, idx_v, rows_v, pos_v, sem):
    cid = lax.axis_index("c")
    sid = lax.axis_index("s")
    wid = sid * NC + cid
    base = wid * PER_W
    ibase = wid * (PER_W // IW)

    pltpu.sync_copy(pos_hbm, pos_v)

    def chunk(g, _):
        off = base + g * C

        pltpu.sync_copy(x_hbm.at[pl.ds(ibase + g * K, K)], idx_v)

        copies = []
        for j in range(K):
            copies.append(
                pltpu.async_copy(
                    tok_hbm.at[idx_v.at[j]],
                    rows_v.at[pl.ds(j * IW, IW)],
                    sem,
                )
            )
        for c in copies:
            c.wait()

        def add_pos(r, _):
            p = lax.rem(off + r, L)
            plsc.addupdate(rows_v.at[r, pl.ds(0, 16)], pos_v[p, pl.ds(0, 16)])
            plsc.addupdate(rows_v.at[r, pl.ds(16, 16)], pos_v[p, pl.ds(16, 16)])
            return 0

        lax.fori_loop(0, C, add_pos, 0)

        pltpu.sync_copy(rows_v, out_hbm.at[pl.ds(off, C)])
        return 0

    lax.fori_loop(0, NCHUNK, chunk, 0)


@jax.jit
def _emb(x_rows, token_table, pos_table):
    mesh = plsc.VectorSubcoreMesh(core_axis_name="c", subcore_axis_name="s")
    return pl.kernel(
        _emb_body,
        out_type=jax.ShapeDtypeStruct((N, D), jnp.float32),
        mesh=mesh,
        scratch_types=[
            pltpu.VMEM((K, IW), jnp.int32),
            pltpu.VMEM((C, D), jnp.float32),
            pltpu.VMEM((L, D), jnp.float32),
            pltpu.SemaphoreType.DMA,
        ],
    )(x_rows, token_table, pos_table)


def kernel(x, token_table, pos_table):
    x_rows = x.astype(jnp.int32).reshape(N // IW, IW)
    out = _emb(x_rows, token_table, pos_table)
    return out.reshape(B, L, D)


# speedup vs baseline: 1.2285x; 1.2285x over previous
import jax
import jax.numpy as jnp
from jax import lax
from jax.experimental import pallas as pl
from jax.experimental.pallas import tpu as pltpu
from jax.experimental.pallas import tpu_sc as plsc

VOCAB = 1_000_000
L = 200
D = 32
B = 4096
N = B * L

NC, NS = 2, 16
NW = NC * NS
PER_W = N // NW

IW = 128
C = 1024
K = C // IW
NCHUNK = PER_W // C


def _emb_body(x_hbm, tok_hbm, pos_hbm, out_hbm, idx_v, rows_v, pos_v, sem):
    cid = lax.axis_index("c")
    sid = lax.axis_index("s")
    wid = sid * NC + cid
    base = wid * PER_W
    ibase = wid * (PER_W // IW)

    pltpu.sync_copy(pos_hbm, pos_v)

    def chunk(g, _):
        off = base + g * C

        pltpu.sync_copy(x_hbm.at[pl.ds(ibase + g * K, K)], idx_v)

        copies = []
        for j in range(K):
            copies.append(
                pltpu.async_copy(
                    tok_hbm.at[idx_v.at[j]],
                    rows_v.at[pl.ds(j * IW, IW)],
                    sem,
                )
            )
        for c in copies:
            c.wait()

        def add_pos(r, _):
            p = lax.rem(off + r, L)
            plsc.addupdate(rows_v.at[r, pl.ds(0, 16)], pos_v[p, pl.ds(0, 16)])
            plsc.addupdate(rows_v.at[r, pl.ds(16, 16)], pos_v[p, pl.ds(16, 16)])
            return 0

        lax.fori_loop(0, C, add_pos, 0)

        pltpu.sync_copy(rows_v, out_hbm.at[pl.ds(off, C)])
        return 0

    lax.fori_loop(0, NCHUNK, chunk, 0)


@jax.jit
def _emb(x_rows, token_table, pos_table):
    mesh = plsc.VectorSubcoreMesh(core_axis_name="c", subcore_axis_name="s")
    return pl.kernel(
        _emb_body,
        out_type=jax.ShapeDtypeStruct((N, D), jnp.float32),
        mesh=mesh,
        compiler_params=pltpu.CompilerParams(use_tc_tiling_on_sc=False),
        scratch_types=[
            pltpu.VMEM((K, IW), jnp.int32),
            pltpu.VMEM((C, D), jnp.float32),
            pltpu.VMEM((L, D), jnp.float32),
            pltpu.SemaphoreType.DMA,
        ],
    )(x_rows, token_table, pos_table)


def kernel(x, token_table, pos_table):
    x_rows = x.astype(jnp.int32).reshape(N // IW, IW)
    out = _emb(x_rows, token_table, pos_table)
    return out.reshape(B, L, D)


# speedup vs baseline: 1.4752x; 1.2009x over previous
import jax
import jax.numpy as jnp
from jax import lax
from jax.experimental import pallas as pl
from jax.experimental.pallas import tpu as pltpu
from jax.experimental.pallas import tpu_sc as plsc

VOCAB = 1_000_000
L = 200
D = 32
B = 4096
N = B * L

NC, NS = 2, 16
NW = NC * NS
PER_W = N // NW

IW = 128
C = 1024
K = C // IW
NCHUNK = PER_W // C
NBUF = 3


def _emb_body(x_hbm, tok_hbm, pos_hbm, out_hbm, idx_v, rows_v, pos_v, gsem, wsem):
    cid = lax.axis_index("c")
    sid = lax.axis_index("s")
    wid = sid * NC + cid
    base = wid * PER_W
    ibase = wid * (PER_W // IW)

    pltpu.sync_copy(pos_hbm, pos_v)

    def fire(g):
        p = lax.rem(g, NBUF)
        pltpu.sync_copy(x_hbm.at[pl.ds(ibase + g * K, K)], idx_v.at[p])
        for j in range(K):
            pltpu.async_copy(
                tok_hbm.at[idx_v.at[p, j]],
                rows_v.at[p, pl.ds(j * IW, IW)],
                gsem.at[p],
            )

    fire(0)

    def chunk(g, _):
        p = lax.rem(g, NBUF)
        off = base + g * C

        @pl.when(g + 1 < NCHUNK)
        def _fire_next():
            p1 = lax.rem(g + 1, NBUF)

            @pl.when(g + 1 >= NBUF)
            def _wait_writeback():
                pltpu.make_async_copy(
                    rows_v.at[p1], out_hbm.at[pl.ds(0, C)], wsem.at[p1]
                ).wait()

            fire(g + 1)

        pltpu.make_async_copy(
            tok_hbm.at[pl.ds(0, C)], rows_v.at[p], gsem.at[p]
        ).wait()

        phi = lax.rem(off, L)

        @plsc.parallel_loop(0, C, unroll=8)
        def _add_pos(r):
            pr = lax.rem(phi + r, L)
            plsc.addupdate(rows_v.at[p, r, pl.ds(0, 16)], pos_v[pr, pl.ds(0, 16)])
            plsc.addupdate(rows_v.at[p, r, pl.ds(16, 16)], pos_v[pr, pl.ds(16, 16)])

        pltpu.async_copy(rows_v.at[p], out_hbm.at[pl.ds(off, C)], wsem.at[p])
        return 0

    lax.fori_loop(0, NCHUNK, chunk, 0)

    for b in range(NBUF):
        pltpu.make_async_copy(
            rows_v.at[b], out_hbm.at[pl.ds(0, C)], wsem.at[b]
        ).wait()


@jax.jit
def _emb(x_rows, token_table, pos_table):
    mesh = plsc.VectorSubcoreMesh(core_axis_name="c", subcore_axis_name="s")
    return pl.kernel(
        _emb_body,
        out_type=jax.ShapeDtypeStruct((N, D), jnp.float32),
        mesh=mesh,
        compiler_params=pltpu.CompilerParams(use_tc_tiling_on_sc=False),
        scratch_types=[
            pltpu.VMEM((NBUF, K, IW), jnp.int32),
            pltpu.VMEM((NBUF, C, D), jnp.float32),
            pltpu.VMEM((L, D), jnp.float32),
            pltpu.SemaphoreType.DMA((NBUF,)),
            pltpu.SemaphoreType.DMA((NBUF,)),
        ],
    )(x_rows, token_table, pos_table)


def kernel(x, token_table, pos_table):
    x_rows = x.astype(jnp.int32).reshape(N // IW, IW)
    out = _emb(x_rows, token_table, pos_table)
    return out.reshape(B, L, D)
